# baseline (speedup 1.0000x reference)
import functools

import jax
import jax.numpy as jnp
from jax import lax
from jax.experimental import pallas as pl
from jax.experimental.pallas import tpu as pltpu

B = 16
NP = 16384
C = 256
K = 64
H = 4
HD = C // H
M = 4
P = B * M


def _bn_rows(x, g, b):
    mu = jnp.mean(x, axis=0, keepdims=True)
    var = jnp.mean((x - mu) ** 2, axis=0, keepdims=True)
    return (x - mu) * lax.rsqrt(var + 1e-5) * g + b


def _epilogue_body(lf_ref, apad_ref, wqkv_ref, pwT_ref, posb_ref, posg_ref,
                   posb2_ref, resw_ref, resb_ref, resg_ref, resb2_ref,
                   gwT_ref, globg_ref, globb2_ref, probw_ref, probg_ref,
                   probb2_ref, out_ref):
    lf = lf_ref[...]
    a = apad_ref[...]

    a3 = a.reshape(B, M, 128)
    gc = jnp.mean(a3, axis=1, keepdims=True)
    rel = (a3 - gc).reshape(P, 128)

    pwT = pwT_ref[...]
    pe = (rel[:, 0:1] * pwT[0:1, :] + rel[:, 1:2] * pwT[1:2, :]
          + rel[:, 2:3] * pwT[2:3, :]) + posb_ref[...]
    pe = _bn_rows(pe, posg_ref[...], posb2_ref[...])

    qkv = lax.dot_general(lf, wqkv_ref[...], (((1,), (1,)), ((), ())),
                          preferred_element_type=jnp.float32)
    q = qkv[:, 0:C] + pe
    k = qkv[:, C:2 * C] + pe
    v = qkv[:, 2 * C:3 * C] + pe

    rp = lax.broadcasted_iota(jnp.int32, (P, P), 0) // M
    cp = lax.broadcasted_iota(jnp.int32, (P, P), 1) // M
    blockmask = rp == cp
    heads = []
    for h in range(H):
        qh = q[:, h * HD:(h + 1) * HD]
        kh = k[:, h * HD:(h + 1) * HD]
        vh = v[:, h * HD:(h + 1) * HD]
        s = lax.dot_general(qh, kh, (((1,), (1,)), ((), ())),
                            preferred_element_type=jnp.float32) * (1.0 / (HD ** 0.5))
        s = jnp.where(blockmask, s, -1e30)
        s = s - jnp.max(s, axis=1, keepdims=True)
        e = jnp.exp(s)
        attn = e / jnp.sum(e, axis=1, keepdims=True)
        heads.append(lax.dot_general(attn, vh, (((1,), (0,)), ((), ())),
                                     preferred_element_type=jnp.float32))
    vout = jnp.concatenate(heads, axis=1)

    res = lax.dot_general(vout, resw_ref[...], (((1,), (1,)), ((), ())),
                          preferred_element_type=jnp.float32) + resb_ref[...]
    res = _bn_rows(res, resg_ref[...], resb2_ref[...])
    lf2 = lf + res

    gwT = gwT_ref[...]
    gf = (a[:, 0:1] * gwT[0:1, :] + a[:, 1:2] * gwT[1:2, :]
          + a[:, 2:3] * gwT[2:3, :])
    gf = _bn_rows(gf, globg_ref[...], globb2_ref[...])
    gf3 = gf.reshape(B, M, C)
    gfm = jnp.broadcast_to(jnp.max(gf3, axis=1, keepdims=True),
                           (B, M, C)).reshape(P, C)

    cat = jnp.concatenate([lf2, gfm], axis=1)
    prob = lax.dot_general(cat, probw_ref[...], (((1,), (1,)), ((), ())),
                           preferred_element_type=jnp.float32)
    out_ref[...] = _bn_rows(prob, probg_ref[...], probb2_ref[...])


def _epilogue(local_feat, a_points, W_qkv, pos_w, pos_b, pos_bn_g, pos_bn_b,
              res_w, res_b, res_bn_g, res_bn_b, glob_w, glob_bn_g, glob_bn_b,
              prob_w, prob_bn_g, prob_bn_b):
    apad = jnp.zeros((P, 128), jnp.float32).at[:, :3].set(
        a_points.reshape(P, 3))
    pwT = jnp.zeros((8, C), jnp.float32).at[:3, :].set(pos_w.T)
    gwT = jnp.zeros((8, C), jnp.float32).at[:3, :].set(glob_w.T)
    row = lambda x: x.reshape(1, -1)
    out = pl.pallas_call(
        _epilogue_body,
        out_shape=jax.ShapeDtypeStruct((P, 9), jnp.float32),
    )(local_feat, apad, W_qkv, pwT, row(pos_b), row(pos_bn_g), row(pos_bn_b),
      res_w, row(res_b), row(res_bn_g), row(res_bn_b), gwT, row(glob_bn_g),
      row(glob_bn_b), prob_w, row(prob_bn_g), row(prob_bn_b))
    return out.reshape(B, M, 9)


def kernel(a_points, sa_x, sa_xyz, xyz_raw, W_qkv, pos_w, pos_b, pos_bn_g,
           pos_bn_b, res_w, res_b, res_bn_g, res_bn_b, glob_w, glob_bn_g,
           glob_bn_b, prob_w, prob_bn_g, prob_bn_b):
    d = -2.0 * jnp.einsum('bsc,bnc->bsn', a_points, sa_xyz)
    d = d + jnp.sum(a_points ** 2, axis=-1)[:, :, None]
    d = d + jnp.sum(sa_xyz ** 2, axis=-1)[:, None, :]
    _, idx = lax.top_k(-d, K)
    local = jax.vmap(lambda pts, ix: pts[ix])(sa_x, idx)
    local_feat = jnp.max(local, axis=2).reshape(P, C)

    return _epilogue(local_feat, a_points, W_qkv, pos_w, pos_b, pos_bn_g,
                     pos_bn_b, res_w, res_b, res_bn_g, res_bn_b, glob_w,
                     glob_bn_g, glob_bn_b, prob_w, prob_bn_g, prob_bn_b)


# speedup vs baseline: 6.0454x; 6.0454x over previous
import functools

import jax
import jax.numpy as jnp
import numpy as np
from jax import lax
from jax.experimental import pallas as pl
from jax.experimental.pallas import tpu as pltpu
from jax.experimental.pallas import tpu_sc as plsc

B = 16
NP = 16384
C = 256
K = 64
H = 4
HD = C // H
M = 4
P = B * M


def _bn_rows(x, g, b):
    mu = jnp.mean(x, axis=0, keepdims=True)
    var = jnp.mean((x - mu) ** 2, axis=0, keepdims=True)
    return (x - mu) * lax.rsqrt(var + 1e-5) * g + b


def _epilogue_body(lf_ref, apad_ref, wqkv_ref, pwT_ref, posb_ref, posg_ref,
                   posb2_ref, resw_ref, resb_ref, resg_ref, resb2_ref,
                   gwT_ref, globg_ref, globb2_ref, probw_ref, probg_ref,
                   probb2_ref, out_ref):
    lf = lf_ref[...]
    a = apad_ref[...]

    a3 = a.reshape(B, M, 128)
    gc = jnp.mean(a3, axis=1, keepdims=True)
    rel = (a3 - gc).reshape(P, 128)

    pwT = pwT_ref[...]
    pe = (rel[:, 0:1] * pwT[0:1, :] + rel[:, 1:2] * pwT[1:2, :]
          + rel[:, 2:3] * pwT[2:3, :]) + posb_ref[...]
    pe = _bn_rows(pe, posg_ref[...], posb2_ref[...])

    qkv = lax.dot_general(lf, wqkv_ref[...], (((1,), (1,)), ((), ())),
                          preferred_element_type=jnp.float32)
    q = qkv[:, 0:C] + pe
    k = qkv[:, C:2 * C] + pe
    v = qkv[:, 2 * C:3 * C] + pe

    rp = lax.broadcasted_iota(jnp.int32, (P, P), 0) // M
    cp = lax.broadcasted_iota(jnp.int32, (P, P), 1) // M
    blockmask = rp == cp
    heads = []
    for h in range(H):
        qh = q[:, h * HD:(h + 1) * HD]
        kh = k[:, h * HD:(h + 1) * HD]
        vh = v[:, h * HD:(h + 1) * HD]
        s = lax.dot_general(qh, kh, (((1,), (1,)), ((), ())),
                            preferred_element_type=jnp.float32) * (1.0 / (HD ** 0.5))
        s = jnp.where(blockmask, s, -1e30)
        s = s - jnp.max(s, axis=1, keepdims=True)
        e = jnp.exp(s)
        attn = e / jnp.sum(e, axis=1, keepdims=True)
        heads.append(lax.dot_general(attn, vh, (((1,), (0,)), ((), ())),
                                     preferred_element_type=jnp.float32))
    vout = jnp.concatenate(heads, axis=1)

    res = lax.dot_general(vout, resw_ref[...], (((1,), (1,)), ((), ())),
                          preferred_element_type=jnp.float32) + resb_ref[...]
    res = _bn_rows(res, resg_ref[...], resb2_ref[...])
    lf2 = lf + res

    gwT = gwT_ref[...]
    gf = (a[:, 0:1] * gwT[0:1, :] + a[:, 1:2] * gwT[1:2, :]
          + a[:, 2:3] * gwT[2:3, :])
    gf = _bn_rows(gf, globg_ref[...], globb2_ref[...])
    gf3 = gf.reshape(B, M, C)
    gfm = jnp.broadcast_to(jnp.max(gf3, axis=1, keepdims=True),
                           (B, M, C)).reshape(P, C)

    cat = jnp.concatenate([lf2, gfm], axis=1)
    prob = lax.dot_general(cat, probw_ref[...], (((1,), (1,)), ((), ())),
                           preferred_element_type=jnp.float32)
    out_ref[...] = _bn_rows(prob, probg_ref[...], probb2_ref[...])


def _epilogue(local_feat, a_points, W_qkv, pos_w, pos_b, pos_bn_g, pos_bn_b,
              res_w, res_b, res_bn_g, res_bn_b, glob_w, glob_bn_g, glob_bn_b,
              prob_w, prob_bn_g, prob_bn_b):
    apad = jnp.zeros((P, 128), jnp.float32).at[:, :3].set(
        a_points.reshape(P, 3))
    pwT = jnp.zeros((8, C), jnp.float32).at[:3, :].set(pos_w.T)
    gwT = jnp.zeros((8, C), jnp.float32).at[:3, :].set(glob_w.T)
    row = lambda x: x.reshape(1, -1)
    out = pl.pallas_call(
        _epilogue_body,
        out_shape=jax.ShapeDtypeStruct((P, 9), jnp.float32),
    )(local_feat, apad, W_qkv, pwT, row(pos_b), row(pos_bn_g), row(pos_bn_b),
      res_w, row(res_b), row(res_bn_g), row(res_bn_b), gwT, row(glob_bn_g),
      row(glob_bn_b), prob_w, row(prob_bn_g), row(prob_bn_b))
    return out.reshape(B, M, 9)


NC = 2
NS = 16
L = 16
NITER = NP // L
PAIRS_PER_W = P // (NC * NS)
IMIN = np.int32(-2147483648)
M7F = np.int32(0x7FFFFFFF)
M16 = np.int32(-65536)


def _round_bf16(v):
    b = plsc.bitcast(v, jnp.int32)
    r = (b + 32767 + ((b >> 16) & 1)) & M16
    return plsc.bitcast(r, jnp.float32)


def _knn_sc_body(xyz_hbm, anch_hbm, sax_hbm, out_hbm, idxout_hbm,
                 xyz_v, ubits_v, hist_v, ilt_v, ieq_v, idx_v, rows_v, feat_v,
                 anch_v, sem):
    wid = lax.axis_index("s") * NC + lax.axis_index("c")
    lane = lax.iota(jnp.int32, L)
    ones = jnp.ones((L,), jnp.int32)
    zeros16 = jnp.zeros((L,), jnp.int32)

    b = wid // 2
    pltpu.sync_copy(xyz_hbm.at[b], xyz_v)
    pltpu.sync_copy(anch_hbm, anch_v)

    for q in range(PAIRS_PER_W):
        p = wid * PAIRS_PER_W + q

        def splat(col):
            return plsc.load_gather(
                anch_v, [jnp.full((L,), p * 4 + col, jnp.int32)])

        a0, a1, a2 = splat(0), splat(1), splat(2)
        aa = (a0 * a0 + a1 * a1) + a2 * a2
        a0r, a1r, a2r = _round_bf16(a0), _round_bf16(a1), _round_bf16(a2)

        def zero_hist(j, carry):
            hist_v[pl.ds(j * L, L)] = zeros16
            return carry

        lax.fori_loop(0, 256, zero_hist, 0)

        def pass0(i, carry):
            x = xyz_v[pl.ds(i * L, L)]
            y = xyz_v[pl.ds(NP + i * L, L)]
            z = xyz_v[pl.ds(2 * NP + i * L, L)]
            dot = ((a0r * _round_bf16(x) + a1r * _round_bf16(y))
                   + a2r * _round_bf16(z))
            xx = (x * x + y * y) + z * z
            d = (-2.0 * dot + aa) + xx
            bits = plsc.bitcast(d, jnp.int32)
            mask_i = ((bits >> 31) & M7F) | IMIN
            key = plsc.bitcast(bits ^ mask_i, jnp.uint32)
            ubits_v[pl.ds(i * L, L)] = key
            dig = (key >> 24).astype(jnp.int32)
            plsc.addupdate_scatter(hist_v, [lane * 256 + dig], ones)
            return carry

        lax.fori_loop(0, NITER, pass0, 0)

        def digit_find(k_rem):
            def chunk(j, carry):
                done, dig_f, below, total = carry
                acc = zeros16
                for l in range(L):
                    acc = acc + hist_v[pl.ds(l * 256 + j * L, L)]
                csum = plsc.cumsum(acc)
                chunk_tot = jnp.sum(acc)
                m = (total + csum) >= k_rem
                ffs = plsc.all_reduce_ffs(m)
                found = jnp.logical_and(done == 0, (total + chunk_tot) >= k_rem)
                below_in = jnp.sum(jnp.where(lane < ffs, acc, 0))
                dig_f = jnp.where(found, j * L + ffs, dig_f)
                below = jnp.where(found, total + below_in, below)
                done = jnp.where(found, ones, done)
                total = total + chunk_tot
                return done, dig_f, below, total

            init = (zeros16, zeros16, zeros16, jnp.int32(0))
            _, dig_f, below, _ = lax.fori_loop(0, 16, chunk, init)
            return dig_f, below

        k_rem = jnp.full((L,), K, jnp.int32)
        dig0, below0 = digit_find(k_rem)
        prefix = dig0.astype(jnp.uint32) << 24
        k_rem = k_rem - below0

        for t in range(1, 4):
            sh = 24 - 8 * t
            lax.fori_loop(0, 256, zero_hist, 0)

            def hist_pass(i, carry, sh=sh, prefix=prefix):
                key = ubits_v[pl.ds(i * L, L)]
                active = (key >> (sh + 8)) == (prefix >> (sh + 8))
                dig = ((key >> sh) & jnp.uint32(0xFF)).astype(jnp.int32)
                plsc.addupdate_scatter(hist_v, [lane * 256 + dig], ones,
                                       mask=active)
                return carry

            lax.fori_loop(0, NITER, hist_pass, 0)
            digt, belowt = digit_find(k_rem)
            prefix = prefix | (digt.astype(jnp.uint32) << sh)
            k_rem = k_rem - belowt

        T = prefix

        def extract(i, carry):
            o_lt, o_eq = carry
            key = ubits_v[pl.ds(i * L, L)]
            n_vec = i * L + lane
            m_lt = key < T
            pos_lt = o_lt + plsc.cumsum(m_lt.astype(jnp.int32)) - 1
            plsc.store_scatter(ilt_v, [pos_lt], n_vec, mask=m_lt)
            m_eq = key == T
            pos_eq = o_eq + plsc.cumsum(m_eq.astype(jnp.int32)) - 1
            plsc.store_scatter(ieq_v, [pos_eq], n_vec,
                               mask=jnp.logical_and(m_eq, pos_eq < 80))
            o_lt = o_lt + plsc.all_reduce_population_count(m_lt)
            o_eq = o_eq + plsc.all_reduce_population_count(m_eq)
            return o_lt, o_eq

        c_lt, _ = lax.fori_loop(0, NITER, extract, (zeros16, zeros16))

        base = jnp.full((L,), b * NP, jnp.int32)
        for qq in range(K // L):
            jv = lane + qq * L
            sel = jv < c_lt
            g1 = plsc.load_gather(ilt_v, [jnp.minimum(jv, 79)])
            g2 = plsc.load_gather(
                ieq_v, [jnp.clip(jv - c_lt, 0, 79)])
            idx_v[pl.ds(qq * L, L)] = jnp.where(sel, g1, g2) + base

        pltpu.async_copy(sax_hbm.at[idx_v], rows_v, sem).wait()

        for j in range(C // L):
            def mp(r, acc, j=j):
                return jnp.maximum(acc, rows_v[r, pl.ds(j * L, L)])

            feat_v[pl.ds(j * L, L)] = lax.fori_loop(
                0, K, mp, jnp.full((L,), -jnp.inf, jnp.float32))

        pltpu.sync_copy(feat_v, out_hbm.at[p])
        pltpu.sync_copy(idx_v, idxout_hbm.at[p])


def _knn_sc(a_points, sa_x, sa_xyz):
    xyz = sa_xyz.transpose(0, 2, 1).reshape(B, 3 * NP)
    anch = jnp.zeros((P, 4), jnp.float32).at[:, :3].set(
        a_points.reshape(P, 3)).reshape(P * 4)
    sax = sa_x.reshape(B * NP, C)
    mesh = plsc.VectorSubcoreMesh(core_axis_name="c", subcore_axis_name="s",
                                  num_cores=NC, num_subcores=NS)
    f = pl.kernel(
        _knn_sc_body,
        out_type=(jax.ShapeDtypeStruct((P, C), jnp.float32),
                  jax.ShapeDtypeStruct((P, K), jnp.int32)),
        mesh=mesh,
        compiler_params=pltpu.CompilerParams(needs_layout_passes=False),
        scratch_types=[
            pltpu.VMEM((3 * NP,), jnp.float32),
            pltpu.VMEM((NP,), jnp.uint32),
            pltpu.VMEM((4096,), jnp.int32),
            pltpu.VMEM((80,), jnp.int32),
            pltpu.VMEM((80,), jnp.int32),
            pltpu.VMEM((K,), jnp.int32),
            pltpu.VMEM((K, C), jnp.float32),
            pltpu.VMEM((C,), jnp.float32),
            pltpu.VMEM((P * 4,), jnp.float32),
            pltpu.SemaphoreType.DMA,
        ],
    )
    return f(xyz, anch, sax)


def kernel(a_points, sa_x, sa_xyz, xyz_raw, W_qkv, pos_w, pos_b, pos_bn_g,
           pos_bn_b, res_w, res_b, res_bn_g, res_bn_b, glob_w, glob_bn_g,
           glob_bn_b, prob_w, prob_bn_g, prob_bn_b):
    local_feat, _ = _knn_sc(a_points, sa_x, sa_xyz)
    return _epilogue(local_feat, a_points, W_qkv, pos_w, pos_b, pos_bn_g,
                     pos_bn_b, res_w, res_b, res_bn_g, res_bn_b, glob_w,
                     glob_bn_g, glob_bn_b, prob_w, prob_bn_g, prob_bn_b)


# speedup vs baseline: 6.8348x; 1.1306x over previous
import functools

import jax
import jax.numpy as jnp
import numpy as np
from jax import lax
from jax.experimental import pallas as pl
from jax.experimental.pallas import tpu as pltpu
from jax.experimental.pallas import tpu_sc as plsc

B = 16
NP = 16384
C = 256
K = 64
H = 4
HD = C // H
M = 4
P = B * M


def _bn_rows(x, g, b):
    mu = jnp.mean(x, axis=0, keepdims=True)
    var = jnp.mean((x - mu) ** 2, axis=0, keepdims=True)
    return (x - mu) * lax.rsqrt(var + 1e-5) * g + b


def _epilogue_body(lf_ref, apad_ref, wqkv_ref, pwT_ref, posb_ref, posg_ref,
                   posb2_ref, resw_ref, resb_ref, resg_ref, resb2_ref,
                   gwT_ref, globg_ref, globb2_ref, probw_ref, probg_ref,
                   probb2_ref, out_ref):
    lf = lf_ref[...]
    a = apad_ref[...]

    a3 = a.reshape(B, M, 128)
    gc = jnp.mean(a3, axis=1, keepdims=True)
    rel = (a3 - gc).reshape(P, 128)

    pwT = pwT_ref[...]
    pe = (rel[:, 0:1] * pwT[0:1, :] + rel[:, 1:2] * pwT[1:2, :]
          + rel[:, 2:3] * pwT[2:3, :]) + posb_ref[...]
    pe = _bn_rows(pe, posg_ref[...], posb2_ref[...])

    qkv = lax.dot_general(lf, wqkv_ref[...], (((1,), (1,)), ((), ())),
                          preferred_element_type=jnp.float32)
    q = qkv[:, 0:C] + pe
    k = qkv[:, C:2 * C] + pe
    v = qkv[:, 2 * C:3 * C] + pe

    rp = lax.broadcasted_iota(jnp.int32, (P, P), 0) // M
    cp = lax.broadcasted_iota(jnp.int32, (P, P), 1) // M
    blockmask = rp == cp
    heads = []
    for h in range(H):
        qh = q[:, h * HD:(h + 1) * HD]
        kh = k[:, h * HD:(h + 1) * HD]
        vh = v[:, h * HD:(h + 1) * HD]
        s = lax.dot_general(qh, kh, (((1,), (1,)), ((), ())),
                            preferred_element_type=jnp.float32) * (1.0 / (HD ** 0.5))
        s = jnp.where(blockmask, s, -1e30)
        s = s - jnp.max(s, axis=1, keepdims=True)
        e = jnp.exp(s)
        attn = e / jnp.sum(e, axis=1, keepdims=True)
        heads.append(lax.dot_general(attn, vh, (((1,), (0,)), ((), ())),
                                     preferred_element_type=jnp.float32))
    vout = jnp.concatenate(heads, axis=1)

    res = lax.dot_general(vout, resw_ref[...], (((1,), (1,)), ((), ())),
                          preferred_element_type=jnp.float32) + resb_ref[...]
    res = _bn_rows(res, resg_ref[...], resb2_ref[...])
    lf2 = lf + res

    gwT = gwT_ref[...]
    gf = (a[:, 0:1] * gwT[0:1, :] + a[:, 1:2] * gwT[1:2, :]
          + a[:, 2:3] * gwT[2:3, :])
    gf = _bn_rows(gf, globg_ref[...], globb2_ref[...])
    gf3 = gf.reshape(B, M, C)
    gfm = jnp.broadcast_to(jnp.max(gf3, axis=1, keepdims=True),
                           (B, M, C)).reshape(P, C)

    cat = jnp.concatenate([lf2, gfm], axis=1)
    prob = lax.dot_general(cat, probw_ref[...], (((1,), (1,)), ((), ())),
                           preferred_element_type=jnp.float32)
    out_ref[...] = _bn_rows(prob, probg_ref[...], probb2_ref[...])


def _epilogue(local_feat, a_points, W_qkv, pos_w, pos_b, pos_bn_g, pos_bn_b,
              res_w, res_b, res_bn_g, res_bn_b, glob_w, glob_bn_g, glob_bn_b,
              prob_w, prob_bn_g, prob_bn_b):
    apad = jnp.zeros((P, 128), jnp.float32).at[:, :3].set(
        a_points.reshape(P, 3))
    pwT = jnp.zeros((8, C), jnp.float32).at[:3, :].set(pos_w.T)
    gwT = jnp.zeros((8, C), jnp.float32).at[:3, :].set(glob_w.T)
    row = lambda x: x.reshape(1, -1)
    out = pl.pallas_call(
        _epilogue_body,
        out_shape=jax.ShapeDtypeStruct((P, 9), jnp.float32),
    )(local_feat, apad, W_qkv, pwT, row(pos_b), row(pos_bn_g), row(pos_bn_b),
      res_w, row(res_b), row(res_bn_g), row(res_bn_b), gwT, row(glob_bn_g),
      row(glob_bn_b), prob_w, row(prob_bn_g), row(prob_bn_b))
    return out.reshape(B, M, 9)


NC = 2
NS = 16
L = 16
NITER = NP // L
PAIRS_PER_W = P // (NC * NS)
IMIN = np.int32(-2147483648)
M7F = np.int32(0x7FFFFFFF)
M16 = np.int32(-65536)


def _round_bf16(v):
    b = plsc.bitcast(v, jnp.int32)
    r = (b + 32767 + ((b >> 16) & 1)) & M16
    return plsc.bitcast(r, jnp.float32)


U = 2
HREG = 4096


def _knn_sc_body(xyz_hbm, anch_hbm, sax_hbm, out_hbm,
                 xyz_v, ubits_v, hist_v, ilt_v, ieq_v, idx_v, rows_v, feat_v,
                 anch_v, sem):
    wid = lax.axis_index("s") * NC + lax.axis_index("c")
    lane = lax.iota(jnp.int32, L)
    ones = jnp.ones((L,), jnp.int32)
    zeros16 = jnp.zeros((L,), jnp.int32)

    b = wid // 2
    pltpu.sync_copy(xyz_hbm.at[b], xyz_v)
    pltpu.sync_copy(anch_hbm, anch_v)

    for q in range(PAIRS_PER_W):
        p = wid * PAIRS_PER_W + q

        def splat(col):
            return plsc.load_gather(
                anch_v, [jnp.full((L,), p * 4 + col, jnp.int32)])

        a0, a1, a2 = splat(0), splat(1), splat(2)
        aa = (a0 * a0 + a1 * a1) + a2 * a2
        a0r, a1r, a2r = _round_bf16(a0), _round_bf16(a1), _round_bf16(a2)

        def zero_hist():
            @plsc.parallel_loop(0, U * HREG // L, unroll=4)
            def zh(j):
                hist_v[pl.ds(j * L, L)] = zeros16

        zero_hist()
        lane3 = lane * 3

        @plsc.parallel_loop(0, NITER, step=U)
        def pass0(i):
            for u in range(U):
                ii = i + u
                base3 = ii * (L * 3) + lane3
                x = plsc.load_gather(xyz_v, [base3])
                y = plsc.load_gather(xyz_v, [base3 + 1])
                z = plsc.load_gather(xyz_v, [base3 + 2])
                dot = ((a0r * _round_bf16(x) + a1r * _round_bf16(y))
                       + a2r * _round_bf16(z))
                xx = (x * x + y * y) + z * z
                d = (-2.0 * dot + aa) + xx
                bits = plsc.bitcast(d, jnp.int32)
                mask_i = ((bits >> 31) & M7F) | IMIN
                key = plsc.bitcast(bits ^ mask_i, jnp.uint32)
                ubits_v[pl.ds(ii * L, L)] = key
                dig = (key >> 24).astype(jnp.int32)
                plsc.addupdate_scatter(
                    hist_v, [u * HREG + lane * 256 + dig], ones)

        def digit_find(k_rem):
            def chunk(j, carry):
                done, dig_f, below, total = carry
                acc = zeros16
                for u in range(U):
                    for l in range(L):
                        acc = acc + hist_v[pl.ds(u * HREG + l * 256 + j * L,
                                                 L)]
                csum = plsc.cumsum(acc)
                chunk_tot = jnp.sum(acc)
                m = (total + csum) >= k_rem
                ffs = plsc.all_reduce_ffs(m)
                found = jnp.logical_and(done == 0, (total + chunk_tot) >= k_rem)
                below_in = jnp.sum(jnp.where(lane < ffs, acc, 0))
                dig_f = jnp.where(found, j * L + ffs, dig_f)
                below = jnp.where(found, total + below_in, below)
                done = jnp.where(found, ones, done)
                total = total + chunk_tot
                return done, dig_f, below, total

            init = (zeros16, zeros16, zeros16, jnp.int32(0))
            _, dig_f, below, _ = lax.fori_loop(0, 16, chunk, init)
            return dig_f, below

        k_rem = jnp.full((L,), K, jnp.int32)
        dig0, below0 = digit_find(k_rem)
        prefix = dig0.astype(jnp.uint32) << 24
        k_rem = k_rem - below0

        for t in range(1, 4):
            sh = 24 - 8 * t
            zero_hist()

            @plsc.parallel_loop(0, NITER, step=U)
            def hist_pass(i, sh=sh, prefix=prefix):
                for u in range(U):
                    ii = i + u
                    key = ubits_v[pl.ds(ii * L, L)]
                    active = (key >> (sh + 8)) == (prefix >> (sh + 8))
                    dig = ((key >> sh) & jnp.uint32(0xFF)).astype(jnp.int32)
                    plsc.addupdate_scatter(
                        hist_v, [u * HREG + lane * 256 + dig], ones,
                        mask=active)

            digt, belowt = digit_find(k_rem)
            prefix = prefix | (digt.astype(jnp.uint32) << sh)
            k_rem = k_rem - belowt

        T = prefix

        @plsc.parallel_loop(0, NITER, carry=(zeros16, zeros16))
        def ext_counts(i, carry):
            o_lt, o_eq = carry
            key = ubits_v[pl.ds(i * L, L)]
            n_vec = i * L + lane
            m_lt = key < T
            pos_lt = o_lt + plsc.cumsum(m_lt.astype(jnp.int32)) - 1
            plsc.store_scatter(ilt_v, [pos_lt], n_vec, mask=m_lt)
            m_eq = key == T
            pos_eq = o_eq + plsc.cumsum(m_eq.astype(jnp.int32)) - 1
            plsc.store_scatter(ieq_v, [pos_eq], n_vec,
                               mask=jnp.logical_and(m_eq, pos_eq < 80))
            o_lt = o_lt + plsc.all_reduce_population_count(m_lt)
            o_eq = o_eq + plsc.all_reduce_population_count(m_eq)
            return o_lt, o_eq

        c_lt, _ = ext_counts

        base = jnp.full((L,), b * NP, jnp.int32)
        for qq in range(K // L):
            jv = lane + qq * L
            sel = jv < c_lt
            g1 = plsc.load_gather(ilt_v, [jnp.minimum(jv, 79)])
            g2 = plsc.load_gather(
                ieq_v, [jnp.clip(jv - c_lt, 0, 79)])
            idx_v[pl.ds(qq * L, L)] = jnp.where(sel, g1, g2) + base

        pltpu.async_copy(sax_hbm.at[idx_v], rows_v, sem).wait()

        ninf = jnp.full((L,), -jnp.inf, jnp.float32)

        @plsc.parallel_loop(0, K, carry=(ninf,) * (C // L))
        def pooled(r, accs):
            return tuple(
                jnp.maximum(accs[j], rows_v[r, pl.ds(j * L, L)])
                for j in range(C // L))

        for j in range(C // L):
            feat_v[pl.ds(j * L, L)] = pooled[j]

        pltpu.sync_copy(feat_v, out_hbm.at[p])


def _knn_sc(a_points, sa_x, sa_xyz):
    xyz = sa_xyz.reshape(B, 3 * NP)
    anch = jnp.zeros((P, 4), jnp.float32).at[:, :3].set(
        a_points.reshape(P, 3)).reshape(P * 4)
    sax = sa_x.reshape(B * NP, C)
    mesh = plsc.VectorSubcoreMesh(core_axis_name="c", subcore_axis_name="s",
                                  num_cores=NC, num_subcores=NS)
    f = pl.kernel(
        _knn_sc_body,
        out_type=jax.ShapeDtypeStruct((P, C), jnp.float32),
        mesh=mesh,
        compiler_params=pltpu.CompilerParams(needs_layout_passes=False),
        scratch_types=[
            pltpu.VMEM((3 * NP,), jnp.float32),
            pltpu.VMEM((NP,), jnp.uint32),
            pltpu.VMEM((U * HREG,), jnp.int32),
            pltpu.VMEM((80,), jnp.int32),
            pltpu.VMEM((80,), jnp.int32),
            pltpu.VMEM((K,), jnp.int32),
            pltpu.VMEM((K, C), jnp.float32),
            pltpu.VMEM((C,), jnp.float32),
            pltpu.VMEM((P * 4,), jnp.float32),
            pltpu.SemaphoreType.DMA,
        ],
    )
    return f(xyz, anch, sax)


def kernel(a_points, sa_x, sa_xyz, xyz_raw, W_qkv, pos_w, pos_b, pos_bn_g,
           pos_bn_b, res_w, res_b, res_bn_g, res_bn_b, glob_w, glob_bn_g,
           glob_bn_b, prob_w, prob_bn_g, prob_bn_b):
    local_feat = _knn_sc(a_points, sa_x, sa_xyz)
    return _epilogue(local_feat, a_points, W_qkv, pos_w, pos_b, pos_bn_g,
                     pos_bn_b, res_w, res_b, res_bn_g, res_bn_b, glob_w,
                     glob_bn_g, glob_bn_b, prob_w, prob_bn_g, prob_bn_b)


# speedup vs baseline: 7.2224x; 1.0567x over previous
import functools

import jax
import jax.numpy as jnp
import numpy as np
from jax import lax
from jax.experimental import pallas as pl
from jax.experimental.pallas import tpu as pltpu
from jax.experimental.pallas import tpu_sc as plsc

B = 16
NP = 16384
C = 256
K = 64
H = 4
HD = C // H
M = 4
P = B * M


def _bn_rows(x, g, b):
    mu = jnp.mean(x, axis=0, keepdims=True)
    var = jnp.mean((x - mu) ** 2, axis=0, keepdims=True)
    return (x - mu) * lax.rsqrt(var + 1e-5) * g + b


def _epilogue_body(lf_ref, apad_ref, wqkv_ref, pwT_ref, posb_ref, posg_ref,
                   posb2_ref, resw_ref, resb_ref, resg_ref, resb2_ref,
                   gwT_ref, globg_ref, globb2_ref, probw_ref, probg_ref,
                   probb2_ref, out_ref):
    lf = lf_ref[...]
    a = apad_ref[...]

    a3 = a.reshape(B, M, 128)
    gc = jnp.mean(a3, axis=1, keepdims=True)
    rel = (a3 - gc).reshape(P, 128)

    pwT = pwT_ref[...]
    pe = (rel[:, 0:1] * pwT[0:1, :] + rel[:, 1:2] * pwT[1:2, :]
          + rel[:, 2:3] * pwT[2:3, :]) + posb_ref[...]
    pe = _bn_rows(pe, posg_ref[...], posb2_ref[...])

    qkv = lax.dot_general(lf, wqkv_ref[...], (((1,), (1,)), ((), ())),
                          preferred_element_type=jnp.float32)
    q = qkv[:, 0:C] + pe
    k = qkv[:, C:2 * C] + pe
    v = qkv[:, 2 * C:3 * C] + pe

    rp = lax.broadcasted_iota(jnp.int32, (P, P), 0) // M
    cp = lax.broadcasted_iota(jnp.int32, (P, P), 1) // M
    blockmask = rp == cp
    heads = []
    for h in range(H):
        qh = q[:, h * HD:(h + 1) * HD]
        kh = k[:, h * HD:(h + 1) * HD]
        vh = v[:, h * HD:(h + 1) * HD]
        s = lax.dot_general(qh, kh, (((1,), (1,)), ((), ())),
                            preferred_element_type=jnp.float32) * (1.0 / (HD ** 0.5))
        s = jnp.where(blockmask, s, -1e30)
        s = s - jnp.max(s, axis=1, keepdims=True)
        e = jnp.exp(s)
        attn = e / jnp.sum(e, axis=1, keepdims=True)
        heads.append(lax.dot_general(attn, vh, (((1,), (0,)), ((), ())),
                                     preferred_element_type=jnp.float32))
    vout = jnp.concatenate(heads, axis=1)

    res = lax.dot_general(vout, resw_ref[...], (((1,), (1,)), ((), ())),
                          preferred_element_type=jnp.float32) + resb_ref[...]
    res = _bn_rows(res, resg_ref[...], resb2_ref[...])
    lf2 = lf + res

    gwT = gwT_ref[...]
    gf = (a[:, 0:1] * gwT[0:1, :] + a[:, 1:2] * gwT[1:2, :]
          + a[:, 2:3] * gwT[2:3, :])
    gf = _bn_rows(gf, globg_ref[...], globb2_ref[...])
    gf3 = gf.reshape(B, M, C)
    gfm = jnp.broadcast_to(jnp.max(gf3, axis=1, keepdims=True),
                           (B, M, C)).reshape(P, C)

    cat = jnp.concatenate([lf2, gfm], axis=1)
    prob = lax.dot_general(cat, probw_ref[...], (((1,), (1,)), ((), ())),
                           preferred_element_type=jnp.float32)
    out_ref[...] = _bn_rows(prob, probg_ref[...], probb2_ref[...])


def _epilogue(local_feat, a_points, W_qkv, pos_w, pos_b, pos_bn_g, pos_bn_b,
              res_w, res_b, res_bn_g, res_bn_b, glob_w, glob_bn_g, glob_bn_b,
              prob_w, prob_bn_g, prob_bn_b):
    apad = jnp.zeros((P, 128), jnp.float32).at[:, :3].set(
        a_points.reshape(P, 3))
    pwT = jnp.zeros((8, C), jnp.float32).at[:3, :].set(pos_w.T)
    gwT = jnp.zeros((8, C), jnp.float32).at[:3, :].set(glob_w.T)
    row = lambda x: x.reshape(1, -1)
    out = pl.pallas_call(
        _epilogue_body,
        out_shape=jax.ShapeDtypeStruct((P, 9), jnp.float32),
    )(local_feat, apad, W_qkv, pwT, row(pos_b), row(pos_bn_g), row(pos_bn_b),
      res_w, row(res_b), row(res_bn_g), row(res_bn_b), gwT, row(glob_bn_g),
      row(glob_bn_b), prob_w, row(prob_bn_g), row(prob_bn_b))
    return out.reshape(B, M, 9)


NC = 2
NS = 16
L = 16
NITER = NP // L
PAIRS_PER_W = P // (NC * NS)
IMIN = np.int32(-2147483648)
M7F = np.int32(0x7FFFFFFF)
M16 = np.int32(-65536)


def _round_bf16(v):
    b = plsc.bitcast(v, jnp.int32)
    r = (b + 32767 + ((b >> 16) & 1)) & M16
    return plsc.bitcast(r, jnp.float32)


U = 2
HREG = 4096


def _knn_sc_body(xyz_hbm, anch_hbm, sax_hbm, out_hbm,
                 xyz_v, ubits_v, hist_v, ilt_v, ieq_v, idx_v, rows_v, feat_v,
                 anch_v, sem):
    wid = lax.axis_index("s") * NC + lax.axis_index("c")
    lane = lax.iota(jnp.int32, L)
    ones = jnp.ones((L,), jnp.int32)
    zeros16 = jnp.zeros((L,), jnp.int32)

    b = wid // 2
    pltpu.sync_copy(xyz_hbm.at[b], xyz_v)
    pltpu.sync_copy(anch_hbm, anch_v)

    for q in range(PAIRS_PER_W):
        p = wid * PAIRS_PER_W + q

        def splat(col):
            return plsc.load_gather(
                anch_v, [jnp.full((L,), p * 4 + col, jnp.int32)])

        a0, a1, a2 = splat(0), splat(1), splat(2)
        aa = (a0 * a0 + a1 * a1) + a2 * a2
        a0r, a1r, a2r = _round_bf16(a0), _round_bf16(a1), _round_bf16(a2)

        def zero_hist():
            @plsc.parallel_loop(0, U * HREG // L, unroll=4)
            def zh(j):
                hist_v[pl.ds(j * L, L)] = zeros16

        zero_hist()
        lane3 = lane * 3

        @plsc.parallel_loop(0, NITER, step=U, unroll=2)
        def pass0(i):
            for u in range(U):
                ii = i + u
                base3 = ii * (L * 3) + lane3
                x = plsc.load_gather(xyz_v, [base3])
                y = plsc.load_gather(xyz_v, [base3 + 1])
                z = plsc.load_gather(xyz_v, [base3 + 2])
                dot = ((a0r * _round_bf16(x) + a1r * _round_bf16(y))
                       + a2r * _round_bf16(z))
                xx = (x * x + y * y) + z * z
                d = (-2.0 * dot + aa) + xx
                bits = plsc.bitcast(d, jnp.int32)
                mask_i = ((bits >> 31) & M7F) | IMIN
                key = plsc.bitcast(bits ^ mask_i, jnp.uint32)
                ubits_v[pl.ds(ii * L, L)] = key
                dig = (key >> 24).astype(jnp.int32)
                plsc.addupdate_scatter(
                    hist_v, [u * HREG + lane * 256 + dig], ones)

        def digit_find(k_rem):
            def chunk(j, carry):
                done, dig_f, below, total = carry
                acc = zeros16
                for u in range(U):
                    for l in range(L):
                        acc = acc + hist_v[pl.ds(u * HREG + l * 256 + j * L,
                                                 L)]
                csum = plsc.cumsum(acc)
                chunk_tot = jnp.sum(acc)
                m = (total + csum) >= k_rem
                ffs = plsc.all_reduce_ffs(m)
                found = jnp.logical_and(done == 0, (total + chunk_tot) >= k_rem)
                below_in = jnp.sum(jnp.where(lane < ffs, acc, 0))
                dig_f = jnp.where(found, j * L + ffs, dig_f)
                below = jnp.where(found, total + below_in, below)
                done = jnp.where(found, ones, done)
                total = total + chunk_tot
                return done, dig_f, below, total

            init = (zeros16, zeros16, zeros16, jnp.int32(0))
            _, dig_f, below, _ = lax.fori_loop(0, 16, chunk, init)
            return dig_f, below

        k_rem = jnp.full((L,), K, jnp.int32)
        dig0, below0 = digit_find(k_rem)
        prefix = dig0.astype(jnp.uint32) << 24
        k_rem = k_rem - below0

        for t in range(1, 4):
            sh = 24 - 8 * t
            zero_hist()

            @plsc.parallel_loop(0, NITER, step=U, unroll=2)
            def hist_pass(i, sh=sh, prefix=prefix):
                for u in range(U):
                    ii = i + u
                    key = ubits_v[pl.ds(ii * L, L)]
                    active = (key >> (sh + 8)) == (prefix >> (sh + 8))
                    dig = ((key >> sh) & jnp.uint32(0xFF)).astype(jnp.int32)
                    plsc.addupdate_scatter(
                        hist_v, [u * HREG + lane * 256 + dig], ones,
                        mask=active)

            digt, belowt = digit_find(k_rem)
            prefix = prefix | (digt.astype(jnp.uint32) << sh)
            k_rem = k_rem - belowt

        T = prefix

        @plsc.parallel_loop(0, NITER, unroll=2, carry=(zeros16, zeros16))
        def ext_counts(i, carry):
            o_lt, o_eq = carry
            key = ubits_v[pl.ds(i * L, L)]
            n_vec = i * L + lane
            m_lt = key < T
            pos_lt = o_lt + plsc.cumsum(m_lt.astype(jnp.int32)) - 1
            plsc.store_scatter(ilt_v, [pos_lt], n_vec, mask=m_lt)
            m_eq = key == T
            pos_eq = o_eq + plsc.cumsum(m_eq.astype(jnp.int32)) - 1
            plsc.store_scatter(ieq_v, [pos_eq], n_vec,
                               mask=jnp.logical_and(m_eq, pos_eq < 80))
            o_lt = o_lt + plsc.all_reduce_population_count(m_lt)
            o_eq = o_eq + plsc.all_reduce_population_count(m_eq)
            return o_lt, o_eq

        c_lt, _ = ext_counts

        base = jnp.full((L,), b * NP, jnp.int32)
        for qq in range(K // L):
            jv = lane + qq * L
            sel = jv < c_lt
            g1 = plsc.load_gather(ilt_v, [jnp.minimum(jv, 79)])
            g2 = plsc.load_gather(
                ieq_v, [jnp.clip(jv - c_lt, 0, 79)])
            idx_v[pl.ds(qq * L, L)] = jnp.where(sel, g1, g2) + base

        pltpu.async_copy(sax_hbm.at[idx_v], rows_v, sem).wait()

        ninf = jnp.full((L,), -jnp.inf, jnp.float32)

        @plsc.parallel_loop(0, K, unroll=2, carry=(ninf,) * (C // L))
        def pooled(r, accs):
            return tuple(
                jnp.maximum(accs[j], rows_v[r, pl.ds(j * L, L)])
                for j in range(C // L))

        for j in range(C // L):
            feat_v[pl.ds(j * L, L)] = pooled[j]

        pltpu.sync_copy(feat_v, out_hbm.at[p])


def _knn_sc(a_points, sa_x, sa_xyz):
    xyz = sa_xyz.reshape(B, 3 * NP)
    anch = jnp.zeros((P, 4), jnp.float32).at[:, :3].set(
        a_points.reshape(P, 3)).reshape(P * 4)
    sax = sa_x.reshape(B * NP, C)
    mesh = plsc.VectorSubcoreMesh(core_axis_name="c", subcore_axis_name="s",
                                  num_cores=NC, num_subcores=NS)
    f = pl.kernel(
        _knn_sc_body,
        out_type=jax.ShapeDtypeStruct((P, C), jnp.float32),
        mesh=mesh,
        compiler_params=pltpu.CompilerParams(needs_layout_passes=False),
        scratch_types=[
            pltpu.VMEM((3 * NP,), jnp.float32),
            pltpu.VMEM((NP,), jnp.uint32),
            pltpu.VMEM((U * HREG,), jnp.int32),
            pltpu.VMEM((80,), jnp.int32),
            pltpu.VMEM((80,), jnp.int32),
            pltpu.VMEM((K,), jnp.int32),
            pltpu.VMEM((K, C), jnp.float32),
            pltpu.VMEM((C,), jnp.float32),
            pltpu.VMEM((P * 4,), jnp.float32),
            pltpu.SemaphoreType.DMA,
        ],
    )
    return f(xyz, anch, sax)


def kernel(a_points, sa_x, sa_xyz, xyz_raw, W_qkv, pos_w, pos_b, pos_bn_g,
           pos_bn_b, res_w, res_b, res_bn_g, res_bn_b, glob_w, glob_bn_g,
           glob_bn_b, prob_w, prob_bn_g, prob_bn_b):
    local_feat = _knn_sc(a_points, sa_x, sa_xyz)
    return _epilogue(local_feat, a_points, W_qkv, pos_w, pos_b, pos_bn_g,
                     pos_bn_b, res_w, res_b, res_bn_g, res_bn_b, glob_w,
                     glob_bn_g, glob_bn_b, prob_w, prob_bn_g, prob_bn_b)
